# baseline (speedup 1.0000x reference)
import functools

import jax
import jax.numpy as jnp
from jax.experimental import pallas as pl
from jax.experimental.pallas import tpu as pltpu

DIM = 1024
HEADS = 8
HEAD_DIM = 128
WINDOW = 512
P_MEM = 16
SEQ = 2048
T_REAL = P_MEM + SEQ
T_PAD = 2176
QBLK = 544
EPS = 1e-5
NEG = -1e30
BF = jnp.bfloat16
F32 = jnp.float32

_VMEM_BIG = 60 * 1024 * 1024


def _cp(sem):
    sem = tuple("arbitrary" for _ in sem)
    return pltpu.CompilerParams(dimension_semantics=sem,
                                vmem_limit_bytes=_VMEM_BIG)


def _mm(x, w):
    return jax.lax.dot_general(x, w, (((1,), (1,)), ((), ())),
                               preferred_element_type=F32)


def _mmb(x, w):
    return jax.lax.dot_general(x.astype(BF), w.astype(BF),
                               (((1,), (1,)), ((), ())),
                               preferred_element_type=F32)


def _mmt(a, b):
    return jax.lax.dot_general(a, b, (((0,), (0,)), ((), ())),
                               preferred_element_type=F32)


def _ln(x, g, b):
    m = jnp.mean(x, axis=-1, keepdims=True)
    c = x - m
    v = jnp.mean(c * c, axis=-1, keepdims=True)
    return c * jax.lax.rsqrt(v + EPS) * g + b


def _row_spec(bt, n):
    return pl.BlockSpec((bt, n), lambda i: (i, 0))


def _full_spec(shape):
    nd = len(shape)
    return pl.BlockSpec(shape, lambda i: (0,) * nd)


def _linear_body(out_dtype, x_ref, w_ref, b_ref, o_ref):
    y = _mmb(x_ref[...], w_ref[...]) + b_ref[...]
    o_ref[...] = y.astype(out_dtype)


def _linear(x, w, b, out_dtype=F32, bt=256):
    t, k = x.shape
    n = w.shape[0]
    return pl.pallas_call(
        functools.partial(_linear_body, out_dtype),
        grid=(t // bt,),
        in_specs=[_row_spec(bt, k), _full_spec((n, k)), _full_spec((1, n))],
        out_specs=_row_spec(bt, n),
        out_shape=jax.ShapeDtypeStruct((t, n), out_dtype),
        compiler_params=_cp(("parallel",)),
    )(x, w, b.reshape(1, n))


def _proj_body(x_ref, w6_ref, b6_ref, w1_ref, b1_ref, w2_ref, b2_ref,
               mo_ref, k_ref, v_ref, aq_ref, ak_ref, av_ref):
    xb = x_ref[...]
    outs = (None, k_ref, v_ref, aq_ref, ak_ref, av_ref)
    q = None
    for j, o in enumerate(outs):
        yj = (_mmb(xb, w6_ref[j * DIM:(j + 1) * DIM, :])
              + b6_ref[:, j * DIM:(j + 1) * DIM])
        if o is None:
            q = yj
        else:
            o[...] = yj.astype(BF)
    h1 = jax.nn.silu(_mmb(q, w1_ref[...]) + b1_ref[...])
    mo_ref[...] = (_mmb(h1, w2_ref[...]) + b2_ref[...]).astype(BF)


def _proj_mem(xp, w6, b6, w1, b1, w2, b2, bt=544):
    t = xp.shape[0]
    n = w6.shape[0]
    return pl.pallas_call(
        _proj_body,
        grid=(t // bt,),
        in_specs=[_row_spec(bt, DIM), _full_spec((n, DIM)), _full_spec((1, n)),
                  _full_spec((DIM, DIM)), _full_spec((1, DIM)),
                  _full_spec((DIM, DIM)), _full_spec((1, DIM))],
        out_specs=tuple(_row_spec(bt, DIM) for _ in range(6)),
        out_shape=tuple(jax.ShapeDtypeStruct((t, DIM), BF) for _ in range(6)),
        compiler_params=_cp(("parallel",)),
    )(xp, w6, b6.reshape(1, n), w1, b1.reshape(1, DIM), w2,
      b2.reshape(1, DIM))


def _swa_body(q_ref, kp_ref, kc_ref, vp_ref, vc_ref, o_ref):
    i = pl.program_id(0)
    scale = BF(HEAD_DIM ** -0.5)
    dm = (jax.lax.broadcasted_iota(jnp.int32, (QBLK, QBLK), 1)
          - jax.lax.broadcasted_iota(jnp.int32, (QBLK, QBLK), 0))
    mskp = dm > QBLK - WINDOW
    mskc = (dm <= 0) & (dm > -WINDOW)
    pbias = jnp.where(i > 0, 0.0, NEG)
    q4 = q_ref[...] * scale
    for h in range(_SWA_HG):
        sl = slice(h * HEAD_DIM, (h + 1) * HEAD_DIM)
        qh = q4[:, sl]
        sp = _mm(qh, kp_ref[:, sl])
        sc = _mm(qh, kc_ref[:, sl])
        sp = jnp.where(mskp, sp, NEG) + pbias
        sc = jnp.where(mskc, sc, NEG)
        m = jnp.maximum(jnp.max(sp, axis=-1, keepdims=True),
                        jnp.max(sc, axis=-1, keepdims=True))
        ep = jnp.exp(sp - m)
        ec = jnp.exp(sc - m)
        l = (jnp.sum(ep, axis=-1, keepdims=True)
             + jnp.sum(ec, axis=-1, keepdims=True))
        ov = (jax.lax.dot_general(ep.astype(BF), vp_ref[:, sl],
                                  (((1,), (0,)), ((), ())),
                                  preferred_element_type=F32)
              + jax.lax.dot_general(ec.astype(BF), vc_ref[:, sl],
                                    (((1,), (0,)), ((), ())),
                                    preferred_element_type=F32))
        o_ref[:, sl] = (ov / l).astype(BF)


_SWA_HG = 8


def _swa(q, k, v):
    cur = pl.BlockSpec((QBLK, DIM), lambda i: (i, 0))
    prev = pl.BlockSpec((QBLK, DIM), lambda i: (jnp.maximum(i - 1, 0), 0))
    return pl.pallas_call(
        _swa_body,
        grid=(T_PAD // QBLK,),
        in_specs=[cur, prev, cur, prev, cur],
        out_specs=cur,
        out_shape=jax.ShapeDtypeStruct((T_PAD, DIM), BF),
        compiler_params=_cp(("parallel",)),
    )(q, k, k, v, v)


_MG_BT = 544
_MG_STEPS = T_PAD // _MG_BT


def _memgrad_body(k_ref, v_ref, w1_ref, b1_ref, w2_ref, b2_ref,
                  aw_ref, ab_ref, tw_ref, tb_ref, o_ref,
                  acc_g, acc_a, acc_t):
    i = pl.program_id(0)

    @pl.when(i == 0)
    def _():
        acc_g[...] = jnp.zeros_like(acc_g)
        acc_a[...] = jnp.zeros_like(acc_a)
        acc_t[...] = jnp.zeros_like(acc_t)

    kb = k_ref[...].astype(F32)
    rows = i * _MG_BT + jax.lax.broadcasted_iota(jnp.int32, (_MG_BT, 1), 0)
    mask = rows < T_REAL
    h1 = jax.nn.silu(_mm(kb, w1_ref[...]) + b1_ref[...])
    pred = _mm(h1, w2_ref[...]) + b2_ref[...]
    dp = jnp.where(mask, pred - v_ref[...].astype(F32), 0.0)
    acc_g[...] += _mmt(dp, h1)

    za = jnp.sum(kb * aw_ref[...], axis=-1, keepdims=True) + ab_ref[...]
    zt = jnp.sum(kb * tw_ref[...], axis=-1, keepdims=True) + tb_ref[...]
    acc_a[...] += jnp.sum(jnp.where(mask, jax.nn.sigmoid(za), 0.0),
                          axis=0, keepdims=True)
    acc_t[...] += jnp.sum(jnp.where(mask, jax.nn.softplus(zt), 0.0),
                          axis=0, keepdims=True)

    @pl.when(i == _MG_STEPS - 1)
    def _():
        alpha = acc_a[0, 0] / T_REAL
        theta = acc_t[0, 0] / T_REAL
        coef = theta * (2.0 / (T_REAL * DIM))
        o_ref[...] = (1.0 - alpha) * w2_ref[...] - coef * acc_g[...]


def _memgrad(k, v, w1, b1, w2, b2, aw, ab, tw, tb):
    return pl.pallas_call(
        _memgrad_body,
        grid=(_MG_STEPS,),
        in_specs=[_row_spec(_MG_BT, DIM), _row_spec(_MG_BT, DIM),
                  _full_spec((DIM, DIM)), _full_spec((1, DIM)),
                  _full_spec((DIM, DIM)), _full_spec((1, DIM)),
                  _full_spec((1, DIM)), _full_spec((1, 1)),
                  _full_spec((1, DIM)), _full_spec((1, 1))],
        out_specs=_full_spec((DIM, DIM)),
        out_shape=jax.ShapeDtypeStruct((DIM, DIM), F32),
        scratch_shapes=[pltpu.VMEM((DIM, DIM), F32),
                        pltpu.VMEM((1, 1), F32),
                        pltpu.VMEM((1, 1), F32)],
        compiler_params=_cp(("arbitrary",)),
    )(k, v, w1, b1.reshape(1, DIM), w2, b2.reshape(1, DIM),
      aw.reshape(1, DIM), ab.reshape(1, 1), tw.reshape(1, DIM),
      tb.reshape(1, 1))


def _gate_body(a_ref, m_ref, wo, bo, g1g, g1b, g2g, g2b, gwa, gwm, gb,
               l1g, l1b, w3, b3, o_ref, lq_ref, lk_ref, lv_ref):
    a0 = _mmb(a_ref[...], wo[...]) + bo[...]
    an = _ln(a0, g1g[...], g1b[...])
    mn = _ln(m_ref[...].astype(F32), g2g[...], g2b[...])
    z = _mmb(an, gwa[...]) + _mmb(mn, gwm[...]) + gb[...]
    g = jax.nn.sigmoid(z)
    h0 = g * an + (1.0 - g) * mn
    o_ref[...] = h0.astype(o_ref.dtype)
    hn = _ln(h0, l1g[...], l1b[...])
    for j, o in enumerate((lq_ref, lk_ref, lv_ref)):
        o[...] = (_mmb(hn, w3[j * DIM:(j + 1) * DIM, :])
                  + b3[:, j * DIM:(j + 1) * DIM]).astype(BF)


def _gate(a, m, wo, bo, g1g, g1b, g2g, g2b, gwa, gwm, gb,
          l1g, l1b, w3, b3, bt=544):
    t = a.shape[0]
    vec = _full_spec((1, DIM))
    return pl.pallas_call(
        _gate_body,
        grid=(t // bt,),
        in_specs=[_row_spec(bt, DIM), _row_spec(bt, DIM),
                  _full_spec((DIM, DIM)), vec, vec, vec, vec, vec,
                  _full_spec((DIM, DIM)), _full_spec((DIM, DIM)), vec,
                  vec, vec, _full_spec((3 * DIM, DIM)), _full_spec((1, 3 * DIM))],
        out_specs=(_row_spec(bt, DIM),) + tuple(_row_spec(bt, DIM)
                                                for _ in range(3)),
        out_shape=(jax.ShapeDtypeStruct((t, DIM), BF),)
        + tuple(jax.ShapeDtypeStruct((t, DIM), BF) for _ in range(3)),
        compiler_params=_cp(("parallel",)),
    )(a, m, wo, bo.reshape(1, DIM), g1g.reshape(1, DIM), g1b.reshape(1, DIM),
      g2g.reshape(1, DIM), g2b.reshape(1, DIM), gwa, gwm, gb.reshape(1, DIM),
      l1g.reshape(1, DIM), l1b.reshape(1, DIM), w3, b3.reshape(1, 3 * DIM))


def _ln_linear_body(nout, x_ref, g_ref, bb_ref, w_ref, b_ref, *o_refs):
    hn = _ln(x_ref[...].astype(F32), g_ref[...], bb_ref[...])
    y = _mmb(hn, w_ref[...]) + b_ref[...]
    for j, o in enumerate(o_refs):
        o[...] = y[:, j * DIM:(j + 1) * DIM].astype(BF)


def _ln_linear(x, g, bb, w, b, nout, bt=544):
    t, k = x.shape
    n = w.shape[0]
    out = pl.pallas_call(
        functools.partial(_ln_linear_body, nout),
        grid=(t // bt,),
        in_specs=[_row_spec(bt, k), _full_spec((1, k)), _full_spec((1, k)),
                  _full_spec((n, k)), _full_spec((1, n))],
        out_specs=tuple(_row_spec(bt, DIM) for _ in range(nout)),
        out_shape=tuple(jax.ShapeDtypeStruct((t, DIM), BF)
                        for _ in range(nout)),
        compiler_params=_cp(("parallel",)),
    )(x, g.reshape(1, k), bb.reshape(1, k), w, b.reshape(1, n))
    return out if nout > 1 else out[0]


def _layer_out_body(a_ref, h_ref, wo, bo, l2g, l2b, w1, b1, w2, b2, o_ref):
    t = h_ref[...].astype(F32) + _mmb(a_ref[...], wo[...]) + bo[...]
    hn = _ln(t, l2g[...], l2b[...])
    f = jax.nn.silu(_mmb(hn, w1[...]) + b1[...])
    o_ref[...] = (t + _mmb(f, w2[...]) + b2[...]).astype(o_ref.dtype)


def _layer_qkv_body(a_ref, h_ref, wo, bo, l2g, l2b, w1, b1, w2, b2,
                    l1g, l1b, w3, b3, o_ref, lq_ref, lk_ref, lv_ref):
    t = h_ref[...].astype(F32) + _mmb(a_ref[...], wo[...]) + bo[...]
    hn = _ln(t, l2g[...], l2b[...])
    f = jax.nn.silu(_mmb(hn, w1[...]) + b1[...])
    hnext = t + _mmb(f, w2[...]) + b2[...]
    o_ref[...] = hnext.astype(o_ref.dtype)
    hn2 = _ln(hnext, l1g[...], l1b[...])
    for j, o in enumerate((lq_ref, lk_ref, lv_ref)):
        o[...] = (_mmb(hn2, w3[j * DIM:(j + 1) * DIM, :])
                  + b3[:, j * DIM:(j + 1) * DIM]).astype(BF)


def _layer_out(a, h, wo, bo, l2g, l2b, w1, b1, w2, b2,
               nxt=None, bt=272):
    t = a.shape[0]
    vec = _full_spec((1, DIM))
    base_in = [_row_spec(bt, DIM), _row_spec(bt, DIM),
               _full_spec((DIM, DIM)), vec, vec, vec,
               _full_spec((4 * DIM, DIM)), _full_spec((1, 4 * DIM)),
               _full_spec((DIM, 4 * DIM)), vec]
    base_args = [a, h, wo, bo.reshape(1, DIM), l2g.reshape(1, DIM),
                 l2b.reshape(1, DIM), w1, b1.reshape(1, 4 * DIM), w2,
                 b2.reshape(1, DIM)]
    if nxt is None:
        return pl.pallas_call(
            _layer_out_body,
            grid=(t // bt,),
            in_specs=base_in,
            out_specs=_row_spec(bt, DIM),
            out_shape=jax.ShapeDtypeStruct((t, DIM), F32),
            compiler_params=_cp(("parallel",)),
        )(*base_args)
    l1g, l1b, w3, b3 = nxt
    return pl.pallas_call(
        _layer_qkv_body,
        grid=(t // bt,),
        in_specs=base_in + [vec, vec, _full_spec((3 * DIM, DIM)),
                            _full_spec((1, 3 * DIM))],
        out_specs=(_row_spec(bt, DIM),) + tuple(_row_spec(bt, DIM)
                                                for _ in range(3)),
        out_shape=(jax.ShapeDtypeStruct((t, DIM), BF),)
        + tuple(jax.ShapeDtypeStruct((t, DIM), BF) for _ in range(3)),
        compiler_params=_cp(("parallel",)),
    )(*base_args, l1g.reshape(1, DIM), l1b.reshape(1, DIM), w3,
      b3.reshape(1, 3 * DIM))


def _final_body(ha_ref, hb_ref, g_ref, bb_ref, w_ref, b_ref, o_ref):
    hb = jnp.concatenate([ha_ref[P_MEM:, :], hb_ref[:P_MEM, :]], axis=0)
    hn = _ln(hb, g_ref[...], bb_ref[...])
    o_ref[...] = _mmb(hn, w_ref[...]) + b_ref[...]


def _final(h, g, bb, w, b, bt=256):
    return pl.pallas_call(
        _final_body,
        grid=(SEQ // bt,),
        in_specs=[pl.BlockSpec((bt, DIM), lambda i: (i, 0)),
                  pl.BlockSpec((bt, DIM), lambda i: (i + 1, 0)),
                  _full_spec((1, DIM)), _full_spec((1, DIM)),
                  _full_spec((DIM, DIM)), _full_spec((1, DIM))],
        out_specs=_row_spec(bt, DIM),
        out_shape=jax.ShapeDtypeStruct((SEQ, DIM), F32),
        compiler_params=_cp(("parallel",)),
    )(h, h, g.reshape(1, DIM), bb.reshape(1, DIM), w, b.reshape(1, DIM))


def kernel(x, params):
    p = params
    x2 = x[0]

    h = _linear(x2, p['emb_w'], p['emb_b'], out_dtype=BF, bt=512)
    xp = jnp.concatenate(
        [p['pmem'].astype(BF), h, jnp.zeros((T_PAD - T_REAL, DIM), BF)],
        axis=0)

    ap = p['attn']
    w6 = jnp.concatenate([p['q_w'], p['k_w'], p['v_w'],
                          ap['wq'], ap['wk'], ap['wv']], axis=0)
    b6 = jnp.concatenate([p['q_b'], p['k_b'], p['v_b'],
                          ap['bq'], ap['bk'], ap['bv']], axis=0)
    mem_out, k, v, aq, ak, av = _proj_mem(
        xp, w6, b6, p['mem_w1'], p['mem_b1'], p['mem_w2'], p['mem_b2'])

    attn_raw = _swa(aq, ak, av)
    mem_state = _memgrad(k, v, p['mem_w1'], p['mem_b1'], p['mem_w2'],
                         p['mem_b2'], p['alpha_w'], p['alpha_b'],
                         p['theta_w'], p['theta_b'])

    lps = p['layers']
    w3s, b3s = [], []
    for lp in lps:
        la = lp['attn']
        w3s.append(jnp.concatenate([la['wq'], la['wk'], la['wv']], axis=0))
        b3s.append(jnp.concatenate([la['bq'], la['bk'], la['bv']], axis=0))

    h, lq, lk, lv = _gate(attn_raw, mem_out, ap['wo'], ap['bo'],
                          p['gn1_g'], p['gn1_b'], p['gn2_g'], p['gn2_b'],
                          p['gate_w'][:, :DIM], p['gate_w'][:, DIM:],
                          p['gate_b'], lps[0]['ln1_g'], lps[0]['ln1_b'],
                          w3s[0], b3s[0])

    for li, lp in enumerate(lps):
        la = lp['attn']
        ar = _swa(lq, lk, lv)
        if li + 1 < len(lps):
            nlp = lps[li + 1]
            h, lq, lk, lv = _layer_out(
                ar, h, la['wo'], la['bo'], lp['ln2_g'], lp['ln2_b'],
                lp['ffn_w1'], lp['ffn_b1'], lp['ffn_w2'], lp['ffn_b2'],
                nxt=(nlp['ln1_g'], nlp['ln1_b'], w3s[li + 1], b3s[li + 1]))
        else:
            h = _layer_out(ar, h, la['wo'], la['bo'], lp['ln2_g'],
                           lp['ln2_b'], lp['ffn_w1'], lp['ffn_b1'],
                           lp['ffn_w2'], lp['ffn_b2'])

    out = _final(h, p['onorm_g'], p['onorm_b'], p['out_w'], p['out_b'])
    return out[None], mem_state


# speedup vs baseline: 1.0023x; 1.0023x over previous
import functools

import jax
import jax.numpy as jnp
from jax.experimental import pallas as pl
from jax.experimental.pallas import tpu as pltpu

DIM = 1024
HEADS = 8
HEAD_DIM = 128
WINDOW = 512
P_MEM = 16
SEQ = 2048
T_REAL = P_MEM + SEQ
T_PAD = 2176
QBLK = 544
EPS = 1e-5
NEG = -1e30
BF = jnp.bfloat16
F32 = jnp.float32

_VMEM_BIG = 60 * 1024 * 1024


def _cp(sem):
    return pltpu.CompilerParams(dimension_semantics=sem,
                                vmem_limit_bytes=_VMEM_BIG)


def _mm(x, w):
    return jax.lax.dot_general(x, w, (((1,), (1,)), ((), ())),
                               preferred_element_type=F32)


def _mmb(x, w):
    return jax.lax.dot_general(x.astype(BF), w.astype(BF),
                               (((1,), (1,)), ((), ())),
                               preferred_element_type=F32)


def _mmt(a, b):
    return jax.lax.dot_general(a, b, (((0,), (0,)), ((), ())),
                               preferred_element_type=F32)


def _ln(x, g, b):
    m = jnp.mean(x, axis=-1, keepdims=True)
    c = x - m
    v = jnp.mean(c * c, axis=-1, keepdims=True)
    return c * jax.lax.rsqrt(v + EPS) * g + b


def _row_spec(bt, n):
    return pl.BlockSpec((bt, n), lambda i: (i, 0))


def _full_spec(shape):
    nd = len(shape)
    return pl.BlockSpec(shape, lambda i: (0,) * nd)


def _linear_body(out_dtype, x_ref, w_ref, b_ref, o_ref):
    y = _mmb(x_ref[...], w_ref[...]) + b_ref[...]
    o_ref[...] = y.astype(out_dtype)


def _linear(x, w, b, out_dtype=F32, bt=256):
    t, k = x.shape
    n = w.shape[0]
    return pl.pallas_call(
        functools.partial(_linear_body, out_dtype),
        grid=(t // bt,),
        in_specs=[_row_spec(bt, k), _full_spec((n, k)), _full_spec((1, n))],
        out_specs=_row_spec(bt, n),
        out_shape=jax.ShapeDtypeStruct((t, n), out_dtype),
        compiler_params=_cp(("parallel",)),
    )(x, w, b.reshape(1, n))


def _proj_body(x_ref, w6_ref, b6_ref, w1_ref, b1_ref, w2_ref, b2_ref,
               mo_ref, k_ref, v_ref, aq_ref, ak_ref, av_ref):
    xb = x_ref[...]
    outs = (None, k_ref, v_ref, aq_ref, ak_ref, av_ref)
    q = None
    for j, o in enumerate(outs):
        yj = (_mmb(xb, w6_ref[j * DIM:(j + 1) * DIM, :])
              + b6_ref[:, j * DIM:(j + 1) * DIM])
        if o is None:
            q = yj
        else:
            o[...] = yj.astype(BF)
    h1 = jax.nn.silu(_mmb(q, w1_ref[...]) + b1_ref[...])
    mo_ref[...] = (_mmb(h1, w2_ref[...]) + b2_ref[...]).astype(BF)


def _proj_mem(xp, w6, b6, w1, b1, w2, b2, bt=544):
    t = xp.shape[0]
    n = w6.shape[0]
    return pl.pallas_call(
        _proj_body,
        grid=(t // bt,),
        in_specs=[_row_spec(bt, DIM), _full_spec((n, DIM)), _full_spec((1, n)),
                  _full_spec((DIM, DIM)), _full_spec((1, DIM)),
                  _full_spec((DIM, DIM)), _full_spec((1, DIM))],
        out_specs=tuple(_row_spec(bt, DIM) for _ in range(6)),
        out_shape=tuple(jax.ShapeDtypeStruct((t, DIM), BF) for _ in range(6)),
        compiler_params=_cp(("parallel",)),
    )(xp, w6, b6.reshape(1, n), w1, b1.reshape(1, DIM), w2,
      b2.reshape(1, DIM))


def _swa_body(q_ref, k_ref, v_ref, o_ref, kp_s, vp_s):
    i = pl.program_id(0)

    @pl.when(i == 0)
    def _():
        kp_s[...] = jnp.zeros_like(kp_s)
        vp_s[...] = jnp.zeros_like(vp_s)

    scale = BF(HEAD_DIM ** -0.5)
    dm = (jax.lax.broadcasted_iota(jnp.int32, (QBLK, QBLK), 1)
          - jax.lax.broadcasted_iota(jnp.int32, (QBLK, QBLK), 0))
    mskp = dm > QBLK - WINDOW
    mskc = (dm <= 0) & (dm > -WINDOW)
    pbias = jnp.where(i > 0, 0.0, NEG)
    q4 = q_ref[...] * scale
    for h in range(HEADS):
        sl = slice(h * HEAD_DIM, (h + 1) * HEAD_DIM)
        qh = q4[:, sl]
        sp = _mm(qh, kp_s[:, sl])
        sc = _mm(qh, k_ref[:, sl])
        sp = jnp.where(mskp, sp, NEG) + pbias
        sc = jnp.where(mskc, sc, NEG)
        m = jnp.maximum(jnp.max(sp, axis=-1, keepdims=True),
                        jnp.max(sc, axis=-1, keepdims=True))
        ep = jnp.exp(sp - m)
        ec = jnp.exp(sc - m)
        l = (jnp.sum(ep, axis=-1, keepdims=True)
             + jnp.sum(ec, axis=-1, keepdims=True))
        ov = (jax.lax.dot_general(ep.astype(BF), vp_s[:, sl],
                                  (((1,), (0,)), ((), ())),
                                  preferred_element_type=F32)
              + jax.lax.dot_general(ec.astype(BF), v_ref[:, sl],
                                    (((1,), (0,)), ((), ())),
                                    preferred_element_type=F32))
        o_ref[:, sl] = (ov / l).astype(BF)
    kp_s[...] = k_ref[...]
    vp_s[...] = v_ref[...]


def _swa(q, k, v):
    cur = pl.BlockSpec((QBLK, DIM), lambda i: (i, 0))
    return pl.pallas_call(
        _swa_body,
        grid=(T_PAD // QBLK,),
        in_specs=[cur, cur, cur],
        out_specs=cur,
        out_shape=jax.ShapeDtypeStruct((T_PAD, DIM), BF),
        scratch_shapes=[pltpu.VMEM((QBLK, DIM), BF),
                        pltpu.VMEM((QBLK, DIM), BF)],
        compiler_params=_cp(("arbitrary",)),
    )(q, k, v)


_MG_BT = 544
_MG_STEPS = T_PAD // _MG_BT


def _memgrad_body(k_ref, v_ref, w1_ref, b1_ref, w2_ref, b2_ref,
                  aw_ref, ab_ref, tw_ref, tb_ref, o_ref,
                  acc_g, acc_a, acc_t):
    i = pl.program_id(0)

    @pl.when(i == 0)
    def _():
        acc_g[...] = jnp.zeros_like(acc_g)
        acc_a[...] = jnp.zeros_like(acc_a)
        acc_t[...] = jnp.zeros_like(acc_t)

    kb = k_ref[...].astype(F32)
    rows = i * _MG_BT + jax.lax.broadcasted_iota(jnp.int32, (_MG_BT, 1), 0)
    mask = rows < T_REAL
    h1 = jax.nn.silu(_mm(kb, w1_ref[...]) + b1_ref[...])
    pred = _mm(h1, w2_ref[...]) + b2_ref[...]
    dp = jnp.where(mask, pred - v_ref[...].astype(F32), 0.0)
    acc_g[...] += _mmt(dp, h1)

    za = jnp.sum(kb * aw_ref[...], axis=-1, keepdims=True) + ab_ref[...]
    zt = jnp.sum(kb * tw_ref[...], axis=-1, keepdims=True) + tb_ref[...]
    acc_a[...] += jnp.sum(jnp.where(mask, jax.nn.sigmoid(za), 0.0),
                          axis=0, keepdims=True)
    acc_t[...] += jnp.sum(jnp.where(mask, jax.nn.softplus(zt), 0.0),
                          axis=0, keepdims=True)

    @pl.when(i == _MG_STEPS - 1)
    def _():
        alpha = acc_a[0, 0] / T_REAL
        theta = acc_t[0, 0] / T_REAL
        coef = theta * (2.0 / (T_REAL * DIM))
        o_ref[...] = (1.0 - alpha) * w2_ref[...] - coef * acc_g[...]


def _memgrad(k, v, w1, b1, w2, b2, aw, ab, tw, tb):
    return pl.pallas_call(
        _memgrad_body,
        grid=(_MG_STEPS,),
        in_specs=[_row_spec(_MG_BT, DIM), _row_spec(_MG_BT, DIM),
                  _full_spec((DIM, DIM)), _full_spec((1, DIM)),
                  _full_spec((DIM, DIM)), _full_spec((1, DIM)),
                  _full_spec((1, DIM)), _full_spec((1, 1)),
                  _full_spec((1, DIM)), _full_spec((1, 1))],
        out_specs=_full_spec((DIM, DIM)),
        out_shape=jax.ShapeDtypeStruct((DIM, DIM), F32),
        scratch_shapes=[pltpu.VMEM((DIM, DIM), F32),
                        pltpu.VMEM((1, 1), F32),
                        pltpu.VMEM((1, 1), F32)],
        compiler_params=_cp(("arbitrary",)),
    )(k, v, w1, b1.reshape(1, DIM), w2, b2.reshape(1, DIM),
      aw.reshape(1, DIM), ab.reshape(1, 1), tw.reshape(1, DIM),
      tb.reshape(1, 1))


def _gate_body(a_ref, m_ref, wo, bo, g1g, g1b, g2g, g2b, gwa, gwm, gb,
               l1g, l1b, w3, b3, o_ref, lq_ref, lk_ref, lv_ref):
    a0 = _mmb(a_ref[...], wo[...]) + bo[...]
    an = _ln(a0, g1g[...], g1b[...])
    mn = _ln(m_ref[...].astype(F32), g2g[...], g2b[...])
    z = _mmb(an, gwa[...]) + _mmb(mn, gwm[...]) + gb[...]
    g = jax.nn.sigmoid(z)
    h0 = g * an + (1.0 - g) * mn
    o_ref[...] = h0.astype(o_ref.dtype)
    hn = _ln(h0, l1g[...], l1b[...])
    for j, o in enumerate((lq_ref, lk_ref, lv_ref)):
        o[...] = (_mmb(hn, w3[j * DIM:(j + 1) * DIM, :])
                  + b3[:, j * DIM:(j + 1) * DIM]).astype(BF)


def _gate(a, m, wo, bo, g1g, g1b, g2g, g2b, gwa, gwm, gb,
          l1g, l1b, w3, b3, bt=544):
    t = a.shape[0]
    vec = _full_spec((1, DIM))
    return pl.pallas_call(
        _gate_body,
        grid=(t // bt,),
        in_specs=[_row_spec(bt, DIM), _row_spec(bt, DIM),
                  _full_spec((DIM, DIM)), vec, vec, vec, vec, vec,
                  _full_spec((DIM, DIM)), _full_spec((DIM, DIM)), vec,
                  vec, vec, _full_spec((3 * DIM, DIM)), _full_spec((1, 3 * DIM))],
        out_specs=(_row_spec(bt, DIM),) + tuple(_row_spec(bt, DIM)
                                                for _ in range(3)),
        out_shape=(jax.ShapeDtypeStruct((t, DIM), BF),)
        + tuple(jax.ShapeDtypeStruct((t, DIM), BF) for _ in range(3)),
        compiler_params=_cp(("parallel",)),
    )(a, m, wo, bo.reshape(1, DIM), g1g.reshape(1, DIM), g1b.reshape(1, DIM),
      g2g.reshape(1, DIM), g2b.reshape(1, DIM), gwa, gwm, gb.reshape(1, DIM),
      l1g.reshape(1, DIM), l1b.reshape(1, DIM), w3, b3.reshape(1, 3 * DIM))


def _ln_linear_body(nout, x_ref, g_ref, bb_ref, w_ref, b_ref, *o_refs):
    hn = _ln(x_ref[...].astype(F32), g_ref[...], bb_ref[...])
    y = _mmb(hn, w_ref[...]) + b_ref[...]
    for j, o in enumerate(o_refs):
        o[...] = y[:, j * DIM:(j + 1) * DIM].astype(BF)


def _ln_linear(x, g, bb, w, b, nout, bt=544):
    t, k = x.shape
    n = w.shape[0]
    out = pl.pallas_call(
        functools.partial(_ln_linear_body, nout),
        grid=(t // bt,),
        in_specs=[_row_spec(bt, k), _full_spec((1, k)), _full_spec((1, k)),
                  _full_spec((n, k)), _full_spec((1, n))],
        out_specs=tuple(_row_spec(bt, DIM) for _ in range(nout)),
        out_shape=tuple(jax.ShapeDtypeStruct((t, DIM), BF)
                        for _ in range(nout)),
        compiler_params=_cp(("parallel",)),
    )(x, g.reshape(1, k), bb.reshape(1, k), w, b.reshape(1, n))
    return out if nout > 1 else out[0]


def _layer_out_body(a_ref, h_ref, wo, bo, l2g, l2b, w1, b1, w2, b2, o_ref):
    t = h_ref[...].astype(F32) + _mmb(a_ref[...], wo[...]) + bo[...]
    hn = _ln(t, l2g[...], l2b[...])
    f = jax.nn.silu(_mmb(hn, w1[...]) + b1[...])
    o_ref[...] = (t + _mmb(f, w2[...]) + b2[...]).astype(o_ref.dtype)


def _layer_qkv_body(a_ref, h_ref, wo, bo, l2g, l2b, w1, b1, w2, b2,
                    l1g, l1b, w3, b3, o_ref, lq_ref, lk_ref, lv_ref):
    t = h_ref[...].astype(F32) + _mmb(a_ref[...], wo[...]) + bo[...]
    hn = _ln(t, l2g[...], l2b[...])
    f = jax.nn.silu(_mmb(hn, w1[...]) + b1[...])
    hnext = t + _mmb(f, w2[...]) + b2[...]
    o_ref[...] = hnext.astype(o_ref.dtype)
    hn2 = _ln(hnext, l1g[...], l1b[...])
    for j, o in enumerate((lq_ref, lk_ref, lv_ref)):
        o[...] = (_mmb(hn2, w3[j * DIM:(j + 1) * DIM, :])
                  + b3[:, j * DIM:(j + 1) * DIM]).astype(BF)


def _layer_out(a, h, wo, bo, l2g, l2b, w1, b1, w2, b2,
               nxt=None, bt=272):
    t = a.shape[0]
    vec = _full_spec((1, DIM))
    base_in = [_row_spec(bt, DIM), _row_spec(bt, DIM),
               _full_spec((DIM, DIM)), vec, vec, vec,
               _full_spec((4 * DIM, DIM)), _full_spec((1, 4 * DIM)),
               _full_spec((DIM, 4 * DIM)), vec]
    base_args = [a, h, wo, bo.reshape(1, DIM), l2g.reshape(1, DIM),
                 l2b.reshape(1, DIM), w1, b1.reshape(1, 4 * DIM), w2,
                 b2.reshape(1, DIM)]
    if nxt is None:
        return pl.pallas_call(
            _layer_out_body,
            grid=(t // bt,),
            in_specs=base_in,
            out_specs=_row_spec(bt, DIM),
            out_shape=jax.ShapeDtypeStruct((t, DIM), F32),
            compiler_params=_cp(("parallel",)),
        )(*base_args)
    l1g, l1b, w3, b3 = nxt
    return pl.pallas_call(
        _layer_qkv_body,
        grid=(t // bt,),
        in_specs=base_in + [vec, vec, _full_spec((3 * DIM, DIM)),
                            _full_spec((1, 3 * DIM))],
        out_specs=(_row_spec(bt, DIM),) + tuple(_row_spec(bt, DIM)
                                                for _ in range(3)),
        out_shape=(jax.ShapeDtypeStruct((t, DIM), BF),)
        + tuple(jax.ShapeDtypeStruct((t, DIM), BF) for _ in range(3)),
        compiler_params=_cp(("parallel",)),
    )(*base_args, l1g.reshape(1, DIM), l1b.reshape(1, DIM), w3,
      b3.reshape(1, 3 * DIM))


def _final_body(ha_ref, hb_ref, g_ref, bb_ref, w_ref, b_ref, o_ref):
    hb = jnp.concatenate([ha_ref[P_MEM:, :], hb_ref[:P_MEM, :]], axis=0)
    hn = _ln(hb, g_ref[...], bb_ref[...])
    o_ref[...] = _mmb(hn, w_ref[...]) + b_ref[...]


def _final(h, g, bb, w, b, bt=256):
    return pl.pallas_call(
        _final_body,
        grid=(SEQ // bt,),
        in_specs=[pl.BlockSpec((bt, DIM), lambda i: (i, 0)),
                  pl.BlockSpec((bt, DIM), lambda i: (i + 1, 0)),
                  _full_spec((1, DIM)), _full_spec((1, DIM)),
                  _full_spec((DIM, DIM)), _full_spec((1, DIM))],
        out_specs=_row_spec(bt, DIM),
        out_shape=jax.ShapeDtypeStruct((SEQ, DIM), F32),
        compiler_params=_cp(("parallel",)),
    )(h, h, g.reshape(1, DIM), bb.reshape(1, DIM), w, b.reshape(1, DIM))


def kernel(x, params):
    p = params
    x2 = x[0]

    h = _linear(x2, p['emb_w'], p['emb_b'], out_dtype=BF, bt=512)
    xp = jnp.concatenate(
        [p['pmem'].astype(BF), h, jnp.zeros((T_PAD - T_REAL, DIM), BF)],
        axis=0)

    ap = p['attn']
    w6 = jnp.concatenate([p['q_w'], p['k_w'], p['v_w'],
                          ap['wq'], ap['wk'], ap['wv']], axis=0)
    b6 = jnp.concatenate([p['q_b'], p['k_b'], p['v_b'],
                          ap['bq'], ap['bk'], ap['bv']], axis=0)
    mem_out, k, v, aq, ak, av = _proj_mem(
        xp, w6, b6, p['mem_w1'], p['mem_b1'], p['mem_w2'], p['mem_b2'])

    attn_raw = _swa(aq, ak, av)
    mem_state = _memgrad(k, v, p['mem_w1'], p['mem_b1'], p['mem_w2'],
                         p['mem_b2'], p['alpha_w'], p['alpha_b'],
                         p['theta_w'], p['theta_b'])

    lps = p['layers']
    w3s, b3s = [], []
    for lp in lps:
        la = lp['attn']
        w3s.append(jnp.concatenate([la['wq'], la['wk'], la['wv']], axis=0))
        b3s.append(jnp.concatenate([la['bq'], la['bk'], la['bv']], axis=0))

    h, lq, lk, lv = _gate(attn_raw, mem_out, ap['wo'], ap['bo'],
                          p['gn1_g'], p['gn1_b'], p['gn2_g'], p['gn2_b'],
                          p['gate_w'][:, :DIM], p['gate_w'][:, DIM:],
                          p['gate_b'], lps[0]['ln1_g'], lps[0]['ln1_b'],
                          w3s[0], b3s[0])

    for li, lp in enumerate(lps):
        la = lp['attn']
        ar = _swa(lq, lk, lv)
        if li + 1 < len(lps):
            nlp = lps[li + 1]
            h, lq, lk, lv = _layer_out(
                ar, h, la['wo'], la['bo'], lp['ln2_g'], lp['ln2_b'],
                lp['ffn_w1'], lp['ffn_b1'], lp['ffn_w2'], lp['ffn_b2'],
                nxt=(nlp['ln1_g'], nlp['ln1_b'], w3s[li + 1], b3s[li + 1]))
        else:
            h = _layer_out(ar, h, la['wo'], la['bo'], lp['ln2_g'],
                           lp['ln2_b'], lp['ffn_w1'], lp['ffn_b1'],
                           lp['ffn_w2'], lp['ffn_b2'])

    out = _final(h, p['onorm_g'], p['onorm_b'], p['out_w'], p['out_b'])
    return out[None], mem_state
